# Initial kernel scaffold; baseline (speedup 1.0000x reference)
import functools

import jax
import jax.numpy as jnp
from jax import lax
from jax.experimental import pallas as pl
from jax.experimental.pallas import tpu as pltpu
from jax.experimental.pallas import tpu_sc as plsc

N = 10000
E = 320000
D = 128
L = 16
NC, NS = 2, 16
NW = NC * NS
RPT = 320
NPAD = RPT * NW
DUMP = RPT
K = 128
CE = 2000
SCAP = 2048
SSZ = SCAP + 2 * K
EPAD = E + 4 * SCAP
BN = 1024

_MESH = plsc.VectorSubcoreMesh(
    core_axis_name="c", subcore_axis_name="s", num_cores=NC, num_subcores=NS)


@functools.partial(
    pl.kernel,
    out_type=(
        jax.ShapeDtypeStruct((NW, EPAD), jnp.int32),
        jax.ShapeDtypeStruct((NW, EPAD), jnp.int32),
        jax.ShapeDtypeStruct((NW, L), jnp.int32),
        jax.ShapeDtypeStruct((NPAD,), jnp.float32),
    ),
    mesh=_MESH,
    scratch_types=(
        pltpu.VMEM((CE,), jnp.int32),
        pltpu.VMEM((CE,), jnp.int32),
        pltpu.VMEM((SSZ,), jnp.int32),
        pltpu.VMEM((SSZ,), jnp.int32),
        pltpu.VMEM((RPT,), jnp.float32),
        pltpu.VMEM((L,), jnp.int32),
    ),
)
def _build_lists(ei, srcs, dstl, meta, cnt, sbuf, dbuf, sstage, dstage,
                 cslab, mbuf):
    wid = lax.axis_index("s") * NC + lax.axis_index("c")
    lo = wid * RPT
    lanes = lax.iota(jnp.int32, L)

    for r in range(RPT // L):
        cslab[pl.ds(r * L, L)] = jnp.zeros((L,), jnp.float32)

    def group(i, carry):
        off, hbm_off = carry
        s16 = sbuf[pl.ds(i * L, L)]
        d16 = dbuf[pl.ds(i * L, L)]
        m = (d16 >= lo) & (d16 < lo + RPT)
        plsc.store_compressed(sstage.at[pl.ds(off, L)], s16, mask=m)
        plsc.store_compressed(dstage.at[pl.ds(off, L)], d16 - lo, mask=m)
        pc = jnp.sum(m.astype(jnp.int32))
        new_off = off + pc

        def count_body(k2, c):
            r = dstage[k2]
            ra = (r >> 4) << 4
            lane = r - ra
            v = cslab[pl.ds(ra, L)]
            cslab[pl.ds(ra, L)] = v + jnp.where(
                lanes == lane, jnp.float32(1.0), jnp.float32(0.0))
            return c
        lax.fori_loop(off, new_off, count_body, 0)

        do_flush = new_off >= SCAP

        @pl.when(do_flush)
        def _():
            pltpu.sync_copy(sstage.at[pl.ds(0, SCAP)],
                            srcs.at[wid, pl.ds(hbm_off, SCAP)])
            pltpu.sync_copy(dstage.at[pl.ds(0, SCAP)],
                            dstl.at[wid, pl.ds(hbm_off, SCAP)])
            ts = sstage[pl.ds(SCAP, L)]
            td = dstage[pl.ds(SCAP, L)]
            sstage[pl.ds(0, L)] = ts
            dstage[pl.ds(0, L)] = td

        off2 = jnp.where(do_flush, new_off - SCAP, new_off)
        hbm2 = jnp.where(do_flush, hbm_off + SCAP, hbm_off)
        return off2, hbm2

    def chunk(c, carry):
        pltpu.sync_copy(ei.at[0, pl.ds(c * CE, CE)], sbuf)
        pltpu.sync_copy(ei.at[1, pl.ds(c * CE, CE)], dbuf)
        return lax.fori_loop(0, CE // L, group, carry)

    off, hbm_off = lax.fori_loop(0, E // CE, chunk,
                                 (jnp.int32(0), jnp.int32(0)))

    total = hbm_off + off
    for j in range(K // L):
        sstage[pl.ds(off + j * L, L)] = jnp.zeros((L,), jnp.int32)
        dstage[pl.ds(off + j * L, L)] = jnp.full((L,), DUMP, jnp.int32)
    pltpu.sync_copy(sstage.at[pl.ds(0, SCAP)],
                    srcs.at[wid, pl.ds(hbm_off, SCAP)])
    pltpu.sync_copy(dstage.at[pl.ds(0, SCAP)],
                    dstl.at[wid, pl.ds(hbm_off, SCAP)])
    pltpu.sync_copy(sstage.at[pl.ds(SCAP, 2 * K)],
                    srcs.at[wid, pl.ds(hbm_off + SCAP, 2 * K)])
    pltpu.sync_copy(dstage.at[pl.ds(SCAP, 2 * K)],
                    dstl.at[wid, pl.ds(hbm_off + SCAP, 2 * K)])
    nch = (total + (K - 1)) // K
    mv = jnp.where(lanes == 1, nch, total)
    mbuf[...] = mv
    pltpu.sync_copy(mbuf, meta.at[wid])
    pltpu.sync_copy(cslab, cnt.at[pl.ds(lo, RPT)])


@functools.partial(
    pl.kernel,
    out_type=jax.ShapeDtypeStruct((NPAD, D), jnp.float32),
    mesh=_MESH,
    scratch_types=(
        pltpu.VMEM((RPT + 8, D), jnp.float32),
        pltpu.VMEM((K,), jnp.int32), pltpu.VMEM((K,), jnp.int32),
        pltpu.VMEM((K,), jnp.int32), pltpu.VMEM((K,), jnp.int32),
        pltpu.VMEM((K, D), jnp.float32), pltpu.VMEM((K, D), jnp.float32),
        pltpu.VMEM((L,), jnp.int32),
        pltpu.SemaphoreType.DMA, pltpu.SemaphoreType.DMA,
    ),
)
def _aggregate(hp, srcs, dstl, meta, y, slab, i0, i1, d0, d1, r0, r1,
               mbuf, s0, s1):
    wid = lax.axis_index("s") * NC + lax.axis_index("c")
    lo = wid * RPT
    pltpu.sync_copy(meta.at[wid], mbuf)
    nch = mbuf[1]
    pltpu.sync_copy(hp.at[pl.ds(lo, RPT)], slab.at[pl.ds(0, RPT)])

    ib = (i0, i1)
    db = (d0, d1)
    rb = (r0, r1)
    sem = (s0, s1)

    def fire(c, b):
        pltpu.sync_copy(srcs.at[wid, pl.ds(c * K, K)], ib[b])
        pltpu.sync_copy(dstl.at[wid, pl.ds(c * K, K)], db[b])
        pltpu.async_copy(hp.at[ib[b]], rb[b], sem[b])

    def wait(b):
        pltpu.make_async_copy(hp.at[ib[b]], rb[b], sem[b]).wait()

    def process(b):
        def body(i, c):
            r = db[b][i]
            for j in range(D // L):
                plsc.addupdate(slab.at[r, pl.ds(j * L, L)],
                               rb[b][i, pl.ds(j * L, L)])
            return c
        lax.fori_loop(0, K, body, 0)

    @pl.when(nch > 0)
    def _():
        fire(0, 0)

    def pair(g2, c):
        c0 = g2 * 2

        @pl.when(c0 + 1 < nch)
        def _():
            fire(c0 + 1, 1)

        wait(0)
        process(0)

        @pl.when(c0 + 2 < nch)
        def _():
            fire(c0 + 2, 0)

        @pl.when(c0 + 1 < nch)
        def _():
            wait(1)
            process(1)

        return c

    lax.fori_loop(0, (nch + 1) // 2, pair, 0)
    pltpu.sync_copy(slab.at[pl.ds(0, RPT)], y.at[pl.ds(lo, RPT)])


def _dis_body(c_ref, o_ref):
    o_ref[...] = lax.rsqrt(c_ref[...] + 1.0)


def _dis(cnt):
    o = pl.pallas_call(
        _dis_body,
        out_shape=jax.ShapeDtypeStruct((NPAD // D, D), jnp.float32),
    )(cnt.reshape(NPAD // D, D))
    return o.reshape(NPAD, 1)


def _mm_body(x_ref, w_ref, b_ref, pre_ref, post_ref, o_ref):
    xb = x_ref[...] * pre_ref[...] + b_ref[...]
    acc = jnp.dot(xb, w_ref[...], preferred_element_type=jnp.float32)
    o_ref[...] = acc * post_ref[...]


def _mm(xp, W, b2, pre, post):
    return pl.pallas_call(
        _mm_body,
        grid=(NPAD // BN,),
        in_specs=[
            pl.BlockSpec((BN, D), lambda i: (i, 0)),
            pl.BlockSpec((D, D), lambda i: (0, 0)),
            pl.BlockSpec((1, D), lambda i: (0, 0)),
            pl.BlockSpec((BN, 1), lambda i: (i, 0)),
            pl.BlockSpec((BN, 1), lambda i: (i, 0)),
        ],
        out_specs=pl.BlockSpec((BN, D), lambda i: (i, 0)),
        out_shape=jax.ShapeDtypeStruct((NPAD, D), jnp.float32),
    )(xp, W, b2, pre, post)


def _fin_body(y_ref, s_ref, b_ref, o_ref):
    o_ref[...] = y_ref[...] * s_ref[...] + b_ref[...]


def _fin(y, dis2, b2):
    return pl.pallas_call(
        _fin_body,
        grid=(NPAD // BN,),
        in_specs=[
            pl.BlockSpec((BN, D), lambda i: (i, 0)),
            pl.BlockSpec((BN, 1), lambda i: (i, 0)),
            pl.BlockSpec((1, D), lambda i: (0, 0)),
        ],
        out_specs=pl.BlockSpec((BN, D), lambda i: (i, 0)),
        out_shape=jax.ShapeDtypeStruct((NPAD, D), jnp.float32),
    )(y, dis2, b2)


def kernel(x, edge_index, W0, b0, W1, b1):
    xp = jnp.concatenate([x, jnp.zeros((NPAD - N, D), x.dtype)], axis=0)
    srcs, dstl, meta, cnt = _build_lists(edge_index)
    dis2 = _dis(cnt)
    ones = jnp.ones((NPAD, 1), jnp.float32)
    zb = jnp.zeros((1, D), jnp.float32)
    h1 = _mm(xp, W0, zb, ones, dis2)
    y1 = _aggregate(h1, srcs, dstl, meta)
    h2 = _mm(y1, W1, b0.reshape(1, D), dis2, dis2)
    y2 = _aggregate(h2, srcs, dstl, meta)
    out = _fin(y2, dis2, b1.reshape(1, D))
    return out[:N]


# speedup vs baseline: 5.2063x; 5.2063x over previous
import functools

import jax
import jax.numpy as jnp
from jax import lax
from jax.experimental import pallas as pl
from jax.experimental.pallas import tpu as pltpu
from jax.experimental.pallas import tpu_sc as plsc

N = 10000
E = 320000
D = 128
L = 16
NC, NS = 2, 16
NW = NC * NS
RPT = 320
NPAD = RPT * NW
DUMP = RPT
SMASK = (1 << 14) - 1
K = 128
CE = 2000
SCAP = 2048
SSZ = SCAP + 2 * K
EPAD = E + 4 * SCAP
BN = 1024

_MESH = plsc.VectorSubcoreMesh(
    core_axis_name="c", subcore_axis_name="s", num_cores=NC, num_subcores=NS)


@functools.partial(
    pl.kernel,
    out_type=(
        jax.ShapeDtypeStruct((NW * EPAD,), jnp.int32),
        jax.ShapeDtypeStruct((NW * L,), jnp.int32),
        jax.ShapeDtypeStruct((NPAD,), jnp.float32),
    ),
    mesh=_MESH,
    compiler_params=pltpu.CompilerParams(needs_layout_passes=False),
    scratch_types=(
        pltpu.VMEM((CE,), jnp.int32),
        pltpu.VMEM((CE,), jnp.int32),
        pltpu.VMEM((SSZ,), jnp.int32),
        pltpu.VMEM((RPT,), jnp.float32),
        pltpu.VMEM((L,), jnp.int32),
    ),
)
def _build_lists(esrc, edst, elist, meta, cnt, sbuf, dbuf, sstage,
                 cslab, mbuf):
    wid = lax.axis_index("s") * NC + lax.axis_index("c")
    lo = wid * RPT
    lbase = wid * EPAD
    lanes = lax.iota(jnp.int32, L)

    for r in range(RPT // L):
        cslab[pl.ds(r * L, L)] = jnp.zeros((L,), jnp.float32)

    def group(i, carry):
        off, hbm_off = carry
        s16 = sbuf[pl.ds(i * L, L)]
        d16 = dbuf[pl.ds(i * L, L)]
        m = (d16 >= lo) & (d16 < lo + RPT)
        packed = s16 | ((d16 - lo) << 14)
        cum = plsc.cumsum(m.astype(jnp.int32))
        plsc.store_scatter(sstage, [off + cum - 1], packed, mask=m)
        pc = cum[L - 1]
        new_off = off + pc

        def count_body(k2, c):
            r = sstage[pl.ds(k2, L)][0] >> 14
            ra = (r >> 4) << 4
            lane = r - ra
            v = cslab[pl.ds(ra, L)]
            cslab[pl.ds(ra, L)] = v + jnp.where(
                lanes == lane, jnp.float32(1.0), jnp.float32(0.0))
            return c
        lax.fori_loop(off, new_off, count_body, 0)

        do_flush = new_off >= SCAP

        @pl.when(do_flush)
        def _():
            ho = pl.multiple_of(lbase + hbm_off, 8)
            pltpu.sync_copy(sstage.at[pl.ds(0, SCAP)],
                            elist.at[pl.ds(ho, SCAP)])
            ts = sstage[pl.ds(SCAP, L)]
            sstage[pl.ds(0, L)] = ts

        off2 = jnp.where(do_flush, new_off - SCAP, new_off)
        hbm2 = jnp.where(do_flush, hbm_off + SCAP, hbm_off)
        return off2, hbm2

    def chunk(c, carry):
        eb = pl.multiple_of(c * CE, 8)
        pltpu.sync_copy(esrc.at[pl.ds(eb, CE)], sbuf)
        pltpu.sync_copy(edst.at[pl.ds(eb, CE)], dbuf)
        return lax.fori_loop(0, CE // L, group, carry)

    off, hbm_off = lax.fori_loop(0, E // CE, chunk,
                                 (jnp.int32(0), jnp.int32(0)))

    total = hbm_off + off
    for j in range(K // L):
        sstage[pl.ds(off + j * L, L)] = jnp.full((L,), DUMP << 14, jnp.int32)
    ho = pl.multiple_of(lbase + hbm_off, 8)
    ho2 = pl.multiple_of(lbase + hbm_off + SCAP, 8)
    pltpu.sync_copy(sstage.at[pl.ds(0, SCAP)], elist.at[pl.ds(ho, SCAP)])
    pltpu.sync_copy(sstage.at[pl.ds(SCAP, 2 * K)],
                    elist.at[pl.ds(ho2, 2 * K)])
    nch = (total + (K - 1)) // K
    mv = jnp.where(lanes == 1, nch, total)
    mbuf[...] = mv
    pltpu.sync_copy(mbuf, meta.at[pl.ds(pl.multiple_of(wid * L, 8), L)])
    pltpu.sync_copy(cslab, cnt.at[pl.ds(pl.multiple_of(lo, 8), RPT)])


@functools.partial(
    pl.kernel,
    out_type=jax.ShapeDtypeStruct((NPAD, D), jnp.float32),
    mesh=_MESH,
    compiler_params=pltpu.CompilerParams(needs_layout_passes=False),
    scratch_types=(
        pltpu.VMEM((RPT + 8, D), jnp.float32),
        pltpu.VMEM((K,), jnp.int32), pltpu.VMEM((K,), jnp.int32),
        pltpu.VMEM((K,), jnp.int32), pltpu.VMEM((K,), jnp.int32),
        pltpu.VMEM((K, D), jnp.float32), pltpu.VMEM((K, D), jnp.float32),
        pltpu.VMEM((L,), jnp.int32),
        pltpu.SemaphoreType.DMA, pltpu.SemaphoreType.DMA,
    ),
)
def _aggregate(hp, elist, meta, y, slab, p0, p1, i0, i1, r0, r1,
               mbuf, s0, s1):
    wid = lax.axis_index("s") * NC + lax.axis_index("c")
    lo = wid * RPT
    lbase = wid * EPAD
    lo8 = pl.multiple_of(lo, 8)
    pltpu.sync_copy(meta.at[pl.ds(pl.multiple_of(wid * L, 8), L)], mbuf)
    nch = mbuf[...][1]
    pltpu.sync_copy(hp.at[pl.ds(lo8, RPT)], slab.at[pl.ds(0, RPT)])

    pb = (p0, p1)
    ib = (i0, i1)
    rb = (r0, r1)
    sem = (s0, s1)

    def fire(c, b):
        co = pl.multiple_of(lbase + c * K, 8)
        pltpu.sync_copy(elist.at[pl.ds(co, K)], pb[b])
        for g in range(K // L):
            w = pb[b][pl.ds(g * L, L)]
            ib[b][pl.ds(g * L, L)] = w & SMASK
        pltpu.async_copy(hp.at[ib[b]], rb[b], sem[b])

    def wait(b):
        pltpu.make_async_copy(hp.at[ib[b]], rb[b], sem[b]).wait()

    def process(b):
        def body(g, c):
            dv = pb[b][pl.ds(g * L, L)] >> 14
            for lane in range(L):
                r = dv[lane]
                for j in range(D // L):
                    plsc.addupdate(slab.at[r, pl.ds(j * L, L)],
                                   rb[b][g * L + lane, pl.ds(j * L, L)])
            return c
        lax.fori_loop(0, K // L, body, 0)

    @pl.when(nch > 0)
    def _():
        fire(0, 0)

    def pair(g2, c):
        c0 = g2 * 2

        @pl.when(c0 + 1 < nch)
        def _():
            fire(c0 + 1, 1)

        wait(0)
        process(0)

        @pl.when(c0 + 2 < nch)
        def _():
            fire(c0 + 2, 0)

        @pl.when(c0 + 1 < nch)
        def _():
            wait(1)
            process(1)

        return c

    lax.fori_loop(0, (nch + 1) // 2, pair, 0)
    pltpu.sync_copy(slab.at[pl.ds(0, RPT)], y.at[pl.ds(lo8, RPT)])


def _dis_body(c_ref, o_ref):
    o_ref[...] = lax.rsqrt(c_ref[...] + 1.0)


def _dis(cnt):
    o = pl.pallas_call(
        _dis_body,
        out_shape=jax.ShapeDtypeStruct((NPAD // D, D), jnp.float32),
    )(cnt.reshape(NPAD // D, D))
    return o.reshape(NPAD, 1)


def _mm_body(x_ref, w_ref, b_ref, pre_ref, post_ref, o_ref):
    xb = x_ref[...] * pre_ref[...] + b_ref[...]
    acc = jnp.dot(xb, w_ref[...], preferred_element_type=jnp.float32)
    o_ref[...] = acc * post_ref[...]


def _mm(xp, W, b2, pre, post):
    return pl.pallas_call(
        _mm_body,
        grid=(NPAD // BN,),
        in_specs=[
            pl.BlockSpec((BN, D), lambda i: (i, 0)),
            pl.BlockSpec((D, D), lambda i: (0, 0)),
            pl.BlockSpec((1, D), lambda i: (0, 0)),
            pl.BlockSpec((BN, 1), lambda i: (i, 0)),
            pl.BlockSpec((BN, 1), lambda i: (i, 0)),
        ],
        out_specs=pl.BlockSpec((BN, D), lambda i: (i, 0)),
        out_shape=jax.ShapeDtypeStruct((NPAD, D), jnp.float32),
    )(xp, W, b2, pre, post)


def _fin_body(y_ref, s_ref, b_ref, o_ref):
    o_ref[...] = y_ref[...] * s_ref[...] + b_ref[...]


def _fin(y, dis2, b2):
    return pl.pallas_call(
        _fin_body,
        grid=(NPAD // BN,),
        in_specs=[
            pl.BlockSpec((BN, D), lambda i: (i, 0)),
            pl.BlockSpec((BN, 1), lambda i: (i, 0)),
            pl.BlockSpec((1, D), lambda i: (0, 0)),
        ],
        out_specs=pl.BlockSpec((BN, D), lambda i: (i, 0)),
        out_shape=jax.ShapeDtypeStruct((NPAD, D), jnp.float32),
    )(y, dis2, b2)


def kernel(x, edge_index, W0, b0, W1, b1):
    xp = jnp.concatenate([x, jnp.zeros((NPAD - N, D), x.dtype)], axis=0)
    elist, meta, cnt = _build_lists(edge_index[0], edge_index[1])
    dis2 = _dis(cnt)
    ones = jnp.ones((NPAD, 1), jnp.float32)
    zb = jnp.zeros((1, D), jnp.float32)
    h1 = _mm(xp, W0, zb, ones, dis2)
    y1 = _aggregate(h1, elist, meta)
    h2 = _mm(y1, W1, b0.reshape(1, D), dis2, dis2)
    y2 = _aggregate(h2, elist, meta)
    out = _fin(y2, dis2, b1.reshape(1, D))
    return out[:N]


# speedup vs baseline: 9.1050x; 1.7488x over previous
import functools

import jax
import jax.numpy as jnp
from jax import lax
from jax.experimental import pallas as pl
from jax.experimental.pallas import tpu as pltpu
from jax.experimental.pallas import tpu_sc as plsc

N = 10000
E = 320000
D = 128
L = 16
NC, NS = 2, 16
NW = NC * NS
RPT = 320
NPAD = RPT * NW
DUMP = RPT
SMASK = (1 << 14) - 1
K = 128
CE = 4000
SCAP = 2048
SSZ = SCAP + 2 * K
EPAD = E + 4 * SCAP
SLABR = RPT + 8
CSL = RPT + L
BN = 1024

_MESH = plsc.VectorSubcoreMesh(
    core_axis_name="c", subcore_axis_name="s", num_cores=NC, num_subcores=NS)


@functools.partial(
    pl.kernel,
    out_type=(
        jax.ShapeDtypeStruct((NW * EPAD,), jnp.int32),
        jax.ShapeDtypeStruct((NW * L,), jnp.int32),
        jax.ShapeDtypeStruct((NPAD,), jnp.float32),
    ),
    mesh=_MESH,
    compiler_params=pltpu.CompilerParams(needs_layout_passes=False),
    scratch_types=(
        pltpu.VMEM((CE,), jnp.int32), pltpu.VMEM((CE,), jnp.int32),
        pltpu.VMEM((CE,), jnp.int32), pltpu.VMEM((CE,), jnp.int32),
        pltpu.VMEM((SSZ,), jnp.int32),
        pltpu.VMEM((RPT + L,), jnp.float32),
        pltpu.VMEM_SHARED((NS * CSL,), jnp.float32),
        pltpu.VMEM((K,), jnp.int32),
        pltpu.VMEM((K,), jnp.int32),
        pltpu.VMEM((K,), jnp.float32),
        pltpu.VMEM((L,), jnp.int32),
        pltpu.SemaphoreType.DMA, pltpu.SemaphoreType.DMA,
    ),
)
def _build_lists(esrc, edst, elist, meta, cnt, sb0, sb1, db0, db1, sstage,
                 zbuf, cshared, cpk, cdl, ones, mbuf, e0, e1):
    wid = lax.axis_index("s") * NC + lax.axis_index("c")
    lo = wid * RPT
    lbase = wid * EPAD
    cbase = pl.multiple_of(lax.axis_index("s") * CSL, 8)
    lanes = lax.iota(jnp.int32, L)

    for r in range(RPT // L + 1):
        zbuf[pl.ds(r * L, L)] = jnp.zeros((L,), jnp.float32)
    for r in range(K // L):
        ones[pl.ds(r * L, L)] = jnp.ones((L,), jnp.float32)
    pltpu.sync_copy(zbuf, cshared.at[pl.ds(cbase, RPT + L)])

    sb = (sb0, sb1)
    db = (db0, db1)
    sem = (e0, e1)

    def efire(c, b):
        eb = pl.multiple_of(c * CE, 8)
        pltpu.async_copy(esrc.at[pl.ds(eb, CE)], sb[b], sem[b])
        pltpu.async_copy(edst.at[pl.ds(eb, CE)], db[b], sem[b])

    def ewait(c, b):
        eb = pl.multiple_of(c * CE, 8)
        pltpu.make_async_copy(esrc.at[pl.ds(eb, CE)], sb[b], sem[b]).wait()
        pltpu.make_async_copy(edst.at[pl.ds(eb, CE)], db[b], sem[b]).wait()

    def group(i, carry, b):
        off, hbm_off = carry
        s16 = sb[b][pl.ds(i * L, L)]
        d16 = db[b][pl.ds(i * L, L)]
        m = (d16 >= lo) & (d16 < lo + RPT)
        pc = plsc.all_reduce_population_count(m)[0]

        @pl.when(pc > 0)
        def _():
            packed = s16 | ((d16 - lo) << 14)
            cum = plsc.cumsum(m.astype(jnp.int32))
            plsc.store_scatter(sstage, [off + cum - 1], packed, mask=m)

        new_off = off + pc
        do_flush = new_off >= SCAP

        @pl.when(do_flush)
        def _():
            ho = pl.multiple_of(lbase + hbm_off, 8)
            pltpu.sync_copy(sstage.at[pl.ds(0, SCAP)],
                            elist.at[pl.ds(ho, SCAP)])
            ts = sstage[pl.ds(SCAP, L)]
            sstage[pl.ds(0, L)] = ts

        off2 = jnp.where(do_flush, new_off - SCAP, new_off)
        hbm2 = jnp.where(do_flush, hbm_off + SCAP, hbm_off)
        return off2, hbm2

    def scan_chunk(c, carry, b):
        ewait(c, b)
        carry = lax.fori_loop(0, CE // L,
                              lambda i, cr: group(i, cr, b), carry)

        @pl.when(c + 2 < E // CE)
        def _():
            efire(c + 2, b)

        return carry

    def pair(p, carry):
        carry = scan_chunk(p * 2, carry, 0)
        carry = scan_chunk(p * 2 + 1, carry, 1)
        return carry

    efire(0, 0)
    efire(1, 1)
    off, hbm_off = lax.fori_loop(0, E // CE // 2, pair,
                                 (jnp.int32(0), jnp.int32(0)))

    total = hbm_off + off
    for j in range(K // L):
        sstage[pl.ds(off + j * L, L)] = jnp.full((L,), DUMP << 14, jnp.int32)
    ho = pl.multiple_of(lbase + hbm_off, 8)
    ho2 = pl.multiple_of(lbase + hbm_off + SCAP, 8)
    pltpu.sync_copy(sstage.at[pl.ds(0, SCAP)], elist.at[pl.ds(ho, SCAP)])
    pltpu.sync_copy(sstage.at[pl.ds(SCAP, 2 * K)],
                    elist.at[pl.ds(ho2, 2 * K)])
    nch = (total + (K - 1)) // K

    def count_chunk(c, carry):
        co = pl.multiple_of(lbase + c * K, 8)
        pltpu.sync_copy(elist.at[pl.ds(co, K)], cpk)
        for g in range(K // L):
            cdl[pl.ds(g * L, L)] = (cpk[pl.ds(g * L, L)] >> 14) + cbase
        pltpu.sync_copy(ones, cshared.at[cdl], add=True)
        return carry

    lax.fori_loop(0, nch, count_chunk, 0)

    mv = jnp.where(lanes == 1, nch, total)
    mbuf[...] = mv
    pltpu.sync_copy(mbuf, meta.at[pl.ds(pl.multiple_of(wid * L, 8), L)])
    pltpu.sync_copy(cshared.at[pl.ds(cbase, RPT)], zbuf.at[pl.ds(0, RPT)])
    pltpu.sync_copy(zbuf.at[pl.ds(0, RPT)],
                    cnt.at[pl.ds(pl.multiple_of(lo, 8), RPT)])


@functools.partial(
    pl.kernel,
    out_type=jax.ShapeDtypeStruct((NPAD, D), jnp.float32),
    mesh=_MESH,
    compiler_params=pltpu.CompilerParams(needs_layout_passes=False),
    scratch_types=(
        pltpu.VMEM_SHARED((NS * SLABR, D), jnp.float32),
        pltpu.VMEM((K,), jnp.int32), pltpu.VMEM((K,), jnp.int32),
        pltpu.VMEM((K,), jnp.int32), pltpu.VMEM((K,), jnp.int32),
        pltpu.VMEM((K,), jnp.int32), pltpu.VMEM((K,), jnp.int32),
        pltpu.VMEM((K, D), jnp.float32), pltpu.VMEM((K, D), jnp.float32),
        pltpu.VMEM((L,), jnp.int32),
        pltpu.SemaphoreType.DMA, pltpu.SemaphoreType.DMA,
    ),
)
def _aggregate(hp, elist, meta, y, slab, p0, p1, i0, i1, d0, d1, r0, r1,
               mbuf, s0, s1):
    wid = lax.axis_index("s") * NC + lax.axis_index("c")
    lo = wid * RPT
    lbase = wid * EPAD
    lo8 = pl.multiple_of(lo, 8)
    sbase = pl.multiple_of(lax.axis_index("s") * SLABR, 8)
    pltpu.sync_copy(meta.at[pl.ds(pl.multiple_of(wid * L, 8), L)], mbuf)
    nch = mbuf[...][1]
    pltpu.sync_copy(hp.at[pl.ds(lo8, RPT)], slab.at[pl.ds(sbase, RPT)])

    pb = (p0, p1)
    ib = (i0, i1)
    dbf = (d0, d1)
    rb = (r0, r1)
    sem = (s0, s1)

    def fire(c, b):
        co = pl.multiple_of(lbase + c * K, 8)
        pltpu.sync_copy(elist.at[pl.ds(co, K)], pb[b])
        for g in range(K // L):
            w = pb[b][pl.ds(g * L, L)]
            ib[b][pl.ds(g * L, L)] = w & SMASK
            dbf[b][pl.ds(g * L, L)] = (w >> 14) + sbase
        pltpu.async_copy(hp.at[ib[b]], rb[b], sem[b])

    def wait(b):
        pltpu.make_async_copy(hp.at[ib[b]], rb[b], sem[b]).wait()

    def process(b):
        pltpu.sync_copy(rb[b], slab.at[dbf[b]], add=True)

    @pl.when(nch > 0)
    def _():
        fire(0, 0)

    def pair(g2, c):
        c0 = g2 * 2

        @pl.when(c0 + 1 < nch)
        def _():
            fire(c0 + 1, 1)

        wait(0)
        process(0)

        @pl.when(c0 + 2 < nch)
        def _():
            fire(c0 + 2, 0)

        @pl.when(c0 + 1 < nch)
        def _():
            wait(1)
            process(1)

        return c

    lax.fori_loop(0, (nch + 1) // 2, pair, 0)
    pltpu.sync_copy(slab.at[pl.ds(sbase, RPT)], y.at[pl.ds(lo8, RPT)])


def _dis_body(c_ref, o_ref):
    o_ref[...] = lax.rsqrt(c_ref[...] + 1.0)


def _dis(cnt):
    o = pl.pallas_call(
        _dis_body,
        out_shape=jax.ShapeDtypeStruct((NPAD // D, D), jnp.float32),
    )(cnt.reshape(NPAD // D, D))
    return o.reshape(NPAD, 1)


def _mm_body(x_ref, w_ref, b_ref, pre_ref, post_ref, o_ref):
    xb = x_ref[...] * pre_ref[...] + b_ref[...]
    acc = jnp.dot(xb, w_ref[...], preferred_element_type=jnp.float32)
    o_ref[...] = acc * post_ref[...]


def _mm(xp, W, b2, pre, post):
    return pl.pallas_call(
        _mm_body,
        grid=(NPAD // BN,),
        in_specs=[
            pl.BlockSpec((BN, D), lambda i: (i, 0)),
            pl.BlockSpec((D, D), lambda i: (0, 0)),
            pl.BlockSpec((1, D), lambda i: (0, 0)),
            pl.BlockSpec((BN, 1), lambda i: (i, 0)),
            pl.BlockSpec((BN, 1), lambda i: (i, 0)),
        ],
        out_specs=pl.BlockSpec((BN, D), lambda i: (i, 0)),
        out_shape=jax.ShapeDtypeStruct((NPAD, D), jnp.float32),
    )(xp, W, b2, pre, post)


def _fin_body(y_ref, s_ref, b_ref, o_ref):
    o_ref[...] = y_ref[...] * s_ref[...] + b_ref[...]


def _fin(y, dis2, b2):
    return pl.pallas_call(
        _fin_body,
        grid=(NPAD // BN,),
        in_specs=[
            pl.BlockSpec((BN, D), lambda i: (i, 0)),
            pl.BlockSpec((BN, 1), lambda i: (i, 0)),
            pl.BlockSpec((1, D), lambda i: (0, 0)),
        ],
        out_specs=pl.BlockSpec((BN, D), lambda i: (i, 0)),
        out_shape=jax.ShapeDtypeStruct((NPAD, D), jnp.float32),
    )(y, dis2, b2)


def kernel(x, edge_index, W0, b0, W1, b1):
    xp = jnp.concatenate([x, jnp.zeros((NPAD - N, D), x.dtype)], axis=0)
    elist, meta, cnt = _build_lists(edge_index[0], edge_index[1])
    dis2 = _dis(cnt)
    ones = jnp.ones((NPAD, 1), jnp.float32)
    zb = jnp.zeros((1, D), jnp.float32)
    h1 = _mm(xp, W0, zb, ones, dis2)
    y1 = _aggregate(h1, elist, meta)
    h2 = _mm(y1, W1, b0.reshape(1, D), dis2, dis2)
    y2 = _aggregate(h2, elist, meta)
    out = _fin(y2, dis2, b1.reshape(1, D))
    return out[:N]


# speedup vs baseline: 10.6546x; 1.1702x over previous
import functools

import jax
import jax.numpy as jnp
from jax import lax
from jax.experimental import pallas as pl
from jax.experimental.pallas import tpu as pltpu
from jax.experimental.pallas import tpu_sc as plsc

N = 10000
E = 320000
D = 128
L = 16
NC, NS = 2, 16
NW = NC * NS
RPT = 320
NPAD = RPT * NW
DUMP = RPT
SMASK = (1 << 14) - 1
K = 128
CE = 4000
SCAP = 2048
SSZ = 8192
SMSK = SSZ - 1
EPAD = E + 4 * SCAP
SLABR = RPT + 8
CSL = RPT + L
BN = 1024

_MESH = plsc.VectorSubcoreMesh(
    core_axis_name="c", subcore_axis_name="s", num_cores=NC, num_subcores=NS)


@functools.partial(
    pl.kernel,
    out_type=(
        jax.ShapeDtypeStruct((NW * EPAD,), jnp.int32),
        jax.ShapeDtypeStruct((NW * L,), jnp.int32),
        jax.ShapeDtypeStruct((NPAD,), jnp.float32),
    ),
    mesh=_MESH,
    compiler_params=pltpu.CompilerParams(needs_layout_passes=False),
    scratch_types=(
        pltpu.VMEM((CE,), jnp.int32), pltpu.VMEM((CE,), jnp.int32),
        pltpu.VMEM((CE,), jnp.int32), pltpu.VMEM((CE,), jnp.int32),
        pltpu.VMEM((SSZ,), jnp.int32),
        pltpu.VMEM((RPT + L,), jnp.float32),
        pltpu.VMEM_SHARED((NS * CSL,), jnp.float32),
        pltpu.VMEM((K,), jnp.int32),
        pltpu.VMEM((K,), jnp.float32),
        pltpu.VMEM((L,), jnp.int32),
        pltpu.SemaphoreType.DMA, pltpu.SemaphoreType.DMA,
    ),
)
def _build_lists(esrc, edst, elist, meta, cnt, sb0, sb1, db0, db1, sstage,
                 zbuf, cshared, cdlrow, ones, mbuf, e0, e1):
    wid = lax.axis_index("s") * NC + lax.axis_index("c")
    lo = wid * RPT
    lbase = wid * EPAD
    cbase = pl.multiple_of(lax.axis_index("s") * CSL, 8)
    lanes = lax.iota(jnp.int32, L)

    for r in range(RPT // L + 1):
        zbuf[pl.ds(r * L, L)] = jnp.zeros((L,), jnp.float32)
    for r in range(K // L):
        ones[pl.ds(r * L, L)] = jnp.ones((L,), jnp.float32)
    pltpu.sync_copy(zbuf, cshared.at[pl.ds(cbase, RPT + L)])

    sb = (sb0, sb1)
    db = (db0, db1)
    sem = (e0, e1)

    def efire(c, b):
        eb = pl.multiple_of(c * CE, 8)
        pltpu.async_copy(esrc.at[pl.ds(eb, CE)], sb[b], sem[b])
        pltpu.async_copy(edst.at[pl.ds(eb, CE)], db[b], sem[b])

    def ewait(c, b):
        eb = pl.multiple_of(c * CE, 8)
        pltpu.make_async_copy(esrc.at[pl.ds(eb, CE)], sb[b], sem[b]).wait()
        pltpu.make_async_copy(edst.at[pl.ds(eb, CE)], db[b], sem[b]).wait()

    def flush_window(hbm_off, nsub):
        wb = pl.multiple_of(hbm_off & SMSK, 8)
        ho = pl.multiple_of(lbase + hbm_off, 8)
        pltpu.sync_copy(sstage.at[pl.ds(wb, SCAP)], elist.at[pl.ds(ho, SCAP)])

        def csub(k2, cc):
            for g in range(K // L):
                cdlrow[pl.ds(g * L, L)] = (
                    sstage[pl.ds(wb + k2 * K + g * L, L)] >> 14) + cbase
            pltpu.sync_copy(ones, cshared.at[cdlrow], add=True)
            return cc
        lax.fori_loop(0, nsub, csub, 0)

    def group(i, off, b):
        s16 = sb[b][pl.ds(i * L, L)]
        d16 = db[b][pl.ds(i * L, L)]
        m = (d16 >= lo) & (d16 < lo + RPT)
        pc = plsc.all_reduce_population_count(m)[0]

        @pl.when(pc > 0)
        def _():
            packed = s16 | ((d16 - lo) << 14)
            cum = plsc.cumsum(m.astype(jnp.int32))
            plsc.store_scatter(sstage, [(off + cum - 1) & SMSK], packed,
                               mask=m)
        return off + pc

    def scan_chunk(c, carry, b):
        off, hbm_off = carry
        ewait(c, b)
        off = lax.fori_loop(0, CE // L, lambda i, o: group(i, o, b), off,
                            unroll=4)

        @pl.when(c + 2 < E // CE)
        def _():
            efire(c + 2, b)

        for _ in range(2):
            do_f = off - hbm_off >= SCAP

            @pl.when(do_f)
            def _(ho=hbm_off):
                flush_window(ho, 16)
            hbm_off = jnp.where(do_f, hbm_off + SCAP, hbm_off)
        return off, hbm_off

    def pair(p, carry):
        carry = scan_chunk(p * 2, carry, 0)
        carry = scan_chunk(p * 2 + 1, carry, 1)
        return carry

    efire(0, 0)
    efire(1, 1)
    off, hbm_off = lax.fori_loop(0, E // CE // 2, pair,
                                 (jnp.int32(0), jnp.int32(0)))

    total = off
    padv = jnp.full((L,), DUMP << 14, jnp.int32)
    for j in range(K // L):
        plsc.store_scatter(sstage, [(off + j * L + lanes) & SMSK], padv)
    nch = (total + (K - 1)) // K
    ptotal = nch * K
    for _ in range(4):
        do_f = hbm_off < ptotal

        @pl.when(do_f)
        def _(ho=hbm_off):
            flush_window(ho, jnp.minimum((ptotal - ho) // K, 16))
        hbm_off = jnp.where(do_f, hbm_off + SCAP, hbm_off)

    mv = jnp.where(lanes == 1, nch, total)
    mbuf[...] = mv
    pltpu.sync_copy(mbuf, meta.at[pl.ds(pl.multiple_of(wid * L, 8), L)])
    pltpu.sync_copy(cshared.at[pl.ds(cbase, RPT)], zbuf.at[pl.ds(0, RPT)])
    pltpu.sync_copy(zbuf.at[pl.ds(0, RPT)],
                    cnt.at[pl.ds(pl.multiple_of(lo, 8), RPT)])


@functools.partial(
    pl.kernel,
    out_type=jax.ShapeDtypeStruct((NPAD, D), jnp.float32),
    mesh=_MESH,
    compiler_params=pltpu.CompilerParams(needs_layout_passes=False),
    scratch_types=(
        pltpu.VMEM_SHARED((NS * SLABR, D), jnp.float32),
        pltpu.VMEM((K,), jnp.int32), pltpu.VMEM((K,), jnp.int32),
        pltpu.VMEM((K,), jnp.int32), pltpu.VMEM((K,), jnp.int32),
        pltpu.VMEM((K,), jnp.int32), pltpu.VMEM((K,), jnp.int32),
        pltpu.VMEM((K, D), jnp.float32), pltpu.VMEM((K, D), jnp.float32),
        pltpu.VMEM((L,), jnp.int32),
        pltpu.SemaphoreType.DMA, pltpu.SemaphoreType.DMA,
    ),
)
def _aggregate(hp, elist, meta, y, slab, p0, p1, i0, i1, d0, d1, r0, r1,
               mbuf, s0, s1):
    wid = lax.axis_index("s") * NC + lax.axis_index("c")
    lo = wid * RPT
    lbase = wid * EPAD
    lo8 = pl.multiple_of(lo, 8)
    sbase = pl.multiple_of(lax.axis_index("s") * SLABR, 8)
    pltpu.sync_copy(meta.at[pl.ds(pl.multiple_of(wid * L, 8), L)], mbuf)
    nch = mbuf[...][1]
    pltpu.sync_copy(hp.at[pl.ds(lo8, RPT)], slab.at[pl.ds(sbase, RPT)])

    pb = (p0, p1)
    ib = (i0, i1)
    dbf = (d0, d1)
    rb = (r0, r1)
    sem = (s0, s1)

    def fire(c, b):
        co = pl.multiple_of(lbase + c * K, 8)
        pltpu.sync_copy(elist.at[pl.ds(co, K)], pb[b])
        for g in range(K // L):
            w = pb[b][pl.ds(g * L, L)]
            ib[b][pl.ds(g * L, L)] = w & SMASK
            dbf[b][pl.ds(g * L, L)] = (w >> 14) + sbase
        pltpu.async_copy(hp.at[ib[b]], rb[b], sem[b])

    def wait(b):
        pltpu.make_async_copy(hp.at[ib[b]], rb[b], sem[b]).wait()

    def process(b):
        pltpu.sync_copy(rb[b], slab.at[dbf[b]], add=True)

    @pl.when(nch > 0)
    def _():
        fire(0, 0)

    def pair(g2, c):
        c0 = g2 * 2

        @pl.when(c0 + 1 < nch)
        def _():
            fire(c0 + 1, 1)

        wait(0)
        process(0)

        @pl.when(c0 + 2 < nch)
        def _():
            fire(c0 + 2, 0)

        @pl.when(c0 + 1 < nch)
        def _():
            wait(1)
            process(1)

        return c

    lax.fori_loop(0, (nch + 1) // 2, pair, 0)
    pltpu.sync_copy(slab.at[pl.ds(sbase, RPT)], y.at[pl.ds(lo8, RPT)])


def _dis_body(c_ref, o_ref):
    o_ref[...] = lax.rsqrt(c_ref[...] + 1.0)


def _dis(cnt):
    o = pl.pallas_call(
        _dis_body,
        out_shape=jax.ShapeDtypeStruct((NPAD // D, D), jnp.float32),
    )(cnt.reshape(NPAD // D, D))
    return o.reshape(NPAD, 1)


def _mm_body(x_ref, w_ref, b_ref, pre_ref, post_ref, o_ref):
    xb = x_ref[...] * pre_ref[...] + b_ref[...]
    acc = jnp.dot(xb, w_ref[...], preferred_element_type=jnp.float32)
    o_ref[...] = acc * post_ref[...]


def _mm(xp, W, b2, pre, post):
    return pl.pallas_call(
        _mm_body,
        grid=(NPAD // BN,),
        in_specs=[
            pl.BlockSpec((BN, D), lambda i: (i, 0)),
            pl.BlockSpec((D, D), lambda i: (0, 0)),
            pl.BlockSpec((1, D), lambda i: (0, 0)),
            pl.BlockSpec((BN, 1), lambda i: (i, 0)),
            pl.BlockSpec((BN, 1), lambda i: (i, 0)),
        ],
        out_specs=pl.BlockSpec((BN, D), lambda i: (i, 0)),
        out_shape=jax.ShapeDtypeStruct((NPAD, D), jnp.float32),
    )(xp, W, b2, pre, post)


def _fin_body(y_ref, s_ref, b_ref, o_ref):
    o_ref[...] = y_ref[...] * s_ref[...] + b_ref[...]


def _fin(y, dis2, b2):
    return pl.pallas_call(
        _fin_body,
        grid=(NPAD // BN,),
        in_specs=[
            pl.BlockSpec((BN, D), lambda i: (i, 0)),
            pl.BlockSpec((BN, 1), lambda i: (i, 0)),
            pl.BlockSpec((1, D), lambda i: (0, 0)),
        ],
        out_specs=pl.BlockSpec((BN, D), lambda i: (i, 0)),
        out_shape=jax.ShapeDtypeStruct((NPAD, D), jnp.float32),
    )(y, dis2, b2)


def kernel(x, edge_index, W0, b0, W1, b1):
    xp = jnp.concatenate([x, jnp.zeros((NPAD - N, D), x.dtype)], axis=0)
    elist, meta, cnt = _build_lists(edge_index[0], edge_index[1])
    dis2 = _dis(cnt)
    ones = jnp.ones((NPAD, 1), jnp.float32)
    zb = jnp.zeros((1, D), jnp.float32)
    h1 = _mm(xp, W0, zb, ones, dis2)
    y1 = _aggregate(h1, elist, meta)
    h2 = _mm(y1, W1, b0.reshape(1, D), dis2, dis2)
    y2 = _aggregate(h2, elist, meta)
    out = _fin(y2, dis2, b1.reshape(1, D))
    return out[:N]


# speedup vs baseline: 14.0064x; 1.3146x over previous
import functools

import jax
import jax.numpy as jnp
from jax import lax
from jax.experimental import pallas as pl
from jax.experimental.pallas import tpu as pltpu
from jax.experimental.pallas import tpu_sc as plsc

N = 10000
E = 320000
D = 128
L = 16
NC, NS = 2, 16
NW = NC * NS
RPT = 320
NPAD = RPT * NW
DUMP = RPT
SMASK = (1 << 14) - 1
K = 128
CE = 4000
SCAP = 2048
SSZ = 8192
SMSK = SSZ - 1
EPAD = E + 4 * SCAP
SLABR = RPT + 8
CSL = RPT + L
BN = 1024

_MESH = plsc.VectorSubcoreMesh(
    core_axis_name="c", subcore_axis_name="s", num_cores=NC, num_subcores=NS)


@functools.partial(
    pl.kernel,
    out_type=(
        jax.ShapeDtypeStruct((NW * EPAD,), jnp.int32),
        jax.ShapeDtypeStruct((NW * L,), jnp.int32),
        jax.ShapeDtypeStruct((NPAD,), jnp.float32),
    ),
    mesh=_MESH,
    compiler_params=pltpu.CompilerParams(needs_layout_passes=False),
    scratch_types=(
        pltpu.VMEM((CE,), jnp.int32), pltpu.VMEM((CE,), jnp.int32),
        pltpu.VMEM((CE,), jnp.int32), pltpu.VMEM((CE,), jnp.int32),
        pltpu.VMEM((SSZ,), jnp.int32),
        pltpu.VMEM((RPT + L,), jnp.float32),
        pltpu.VMEM_SHARED((NS * CSL,), jnp.float32),
        pltpu.VMEM((K,), jnp.int32),
        pltpu.VMEM((K,), jnp.float32),
        pltpu.VMEM((L,), jnp.int32),
        pltpu.SemaphoreType.DMA, pltpu.SemaphoreType.DMA,
    ),
)
def _build_lists(esrc, edst, elist, meta, cnt, sb0, sb1, db0, db1, sstage,
                 zbuf, cshared, cdlrow, ones, mbuf, e0, e1):
    wid = lax.axis_index("s") * NC + lax.axis_index("c")
    lo = wid * RPT
    lbase = wid * EPAD
    cbase = pl.multiple_of(lax.axis_index("s") * CSL, 8)
    lanes = lax.iota(jnp.int32, L)

    for r in range(RPT // L + 1):
        zbuf[pl.ds(r * L, L)] = jnp.zeros((L,), jnp.float32)
    for r in range(K // L):
        ones[pl.ds(r * L, L)] = jnp.ones((L,), jnp.float32)
    pltpu.sync_copy(zbuf, cshared.at[pl.ds(cbase, RPT + L)])

    sb = (sb0, sb1)
    db = (db0, db1)
    sem = (e0, e1)

    def efire(c, b):
        eb = pl.multiple_of(c * CE, 8)
        pltpu.async_copy(esrc.at[pl.ds(eb, CE)], sb[b], sem[b])
        pltpu.async_copy(edst.at[pl.ds(eb, CE)], db[b], sem[b])

    def ewait(c, b):
        eb = pl.multiple_of(c * CE, 8)
        pltpu.make_async_copy(esrc.at[pl.ds(eb, CE)], sb[b], sem[b]).wait()
        pltpu.make_async_copy(edst.at[pl.ds(eb, CE)], db[b], sem[b]).wait()

    def flush_window(hbm_off, nsub):
        wb = pl.multiple_of(hbm_off & SMSK, 8)
        ho = pl.multiple_of(lbase + hbm_off, 8)
        pltpu.sync_copy(sstage.at[pl.ds(wb, SCAP)], elist.at[pl.ds(ho, SCAP)])

        def csub(k2, cc):
            for g in range(K // L):
                cdlrow[pl.ds(g * L, L)] = (
                    sstage[pl.ds(wb + k2 * K + g * L, L)] >> 14) + cbase
            pltpu.sync_copy(ones, cshared.at[cdlrow], add=True)
            return cc
        lax.fori_loop(0, nsub, csub, 0)

    def group(i, off, b):
        s16 = sb[b][pl.ds(i * L, L)]
        d16 = db[b][pl.ds(i * L, L)]
        m = (d16 >= lo) & (d16 < lo + RPT)
        packed = s16 | ((d16 - lo) << 14)
        cum = plsc.cumsum(m.astype(jnp.int32))
        plsc.store_scatter(sstage, [(off + cum - 1) & SMSK], packed, mask=m)
        return off + cum[L - 1]

    def scan_chunk(c, carry, b):
        off, hbm_off = carry
        ewait(c, b)
        off = lax.fori_loop(0, CE // L, lambda i, o: group(i, o, b), off,
                            unroll=4)

        @pl.when(c + 2 < E // CE)
        def _():
            efire(c + 2, b)

        for _ in range(2):
            do_f = off - hbm_off >= SCAP

            @pl.when(do_f)
            def _(ho=hbm_off):
                flush_window(ho, 16)
            hbm_off = jnp.where(do_f, hbm_off + SCAP, hbm_off)
        return off, hbm_off

    def pair(p, carry):
        carry = scan_chunk(p * 2, carry, 0)
        carry = scan_chunk(p * 2 + 1, carry, 1)
        return carry

    efire(0, 0)
    efire(1, 1)
    off, hbm_off = lax.fori_loop(0, E // CE // 2, pair,
                                 (jnp.int32(0), jnp.int32(0)))

    total = off
    padv = jnp.full((L,), DUMP << 14, jnp.int32)
    for j in range(K // L):
        plsc.store_scatter(sstage, [(off + j * L + lanes) & SMSK], padv)
    nch = (total + (K - 1)) // K
    ptotal = nch * K
    for _ in range(4):
        do_f = hbm_off < ptotal

        @pl.when(do_f)
        def _(ho=hbm_off):
            flush_window(ho, jnp.minimum((ptotal - ho) // K, 16))
        hbm_off = jnp.where(do_f, hbm_off + SCAP, hbm_off)

    mv = jnp.where(lanes == 1, nch, total)
    mbuf[...] = mv
    pltpu.sync_copy(mbuf, meta.at[pl.ds(pl.multiple_of(wid * L, 8), L)])
    pltpu.sync_copy(cshared.at[pl.ds(cbase, RPT)], zbuf.at[pl.ds(0, RPT)])
    pltpu.sync_copy(zbuf.at[pl.ds(0, RPT)],
                    cnt.at[pl.ds(pl.multiple_of(lo, 8), RPT)])


@functools.partial(
    pl.kernel,
    out_type=jax.ShapeDtypeStruct((NPAD, D), jnp.float32),
    mesh=_MESH,
    compiler_params=pltpu.CompilerParams(needs_layout_passes=False),
    scratch_types=(
        pltpu.VMEM_SHARED((NS * SLABR, D), jnp.float32),
        pltpu.VMEM((K,), jnp.int32), pltpu.VMEM((K,), jnp.int32),
        pltpu.VMEM((K,), jnp.int32), pltpu.VMEM((K,), jnp.int32),
        pltpu.VMEM((K,), jnp.int32), pltpu.VMEM((K,), jnp.int32),
        pltpu.VMEM((K, D), jnp.float32), pltpu.VMEM((K, D), jnp.float32),
        pltpu.VMEM((L,), jnp.int32),
        pltpu.SemaphoreType.DMA, pltpu.SemaphoreType.DMA,
    ),
)
def _aggregate(hp, elist, meta, y, slab, p0, p1, i0, i1, d0, d1, r0, r1,
               mbuf, s0, s1):
    wid = lax.axis_index("s") * NC + lax.axis_index("c")
    lo = wid * RPT
    lbase = wid * EPAD
    lo8 = pl.multiple_of(lo, 8)
    sbase = pl.multiple_of(lax.axis_index("s") * SLABR, 8)
    pltpu.sync_copy(meta.at[pl.ds(pl.multiple_of(wid * L, 8), L)], mbuf)
    nch = mbuf[...][1]
    pltpu.sync_copy(hp.at[pl.ds(lo8, RPT)], slab.at[pl.ds(sbase, RPT)])

    pb = (p0, p1)
    ib = (i0, i1)
    dbf = (d0, d1)
    rb = (r0, r1)
    sem = (s0, s1)

    def fire(c, b):
        co = pl.multiple_of(lbase + c * K, 8)
        pltpu.sync_copy(elist.at[pl.ds(co, K)], pb[b])
        for g in range(K // L):
            w = pb[b][pl.ds(g * L, L)]
            ib[b][pl.ds(g * L, L)] = w & SMASK
            dbf[b][pl.ds(g * L, L)] = (w >> 14) + sbase
        pltpu.async_copy(hp.at[ib[b]], rb[b], sem[b])

    def wait(b):
        pltpu.make_async_copy(hp.at[ib[b]], rb[b], sem[b]).wait()

    def process(b):
        pltpu.sync_copy(rb[b], slab.at[dbf[b]], add=True)

    @pl.when(nch > 0)
    def _():
        fire(0, 0)

    def pair(g2, c):
        c0 = g2 * 2

        @pl.when(c0 + 1 < nch)
        def _():
            fire(c0 + 1, 1)

        wait(0)
        process(0)

        @pl.when(c0 + 2 < nch)
        def _():
            fire(c0 + 2, 0)

        @pl.when(c0 + 1 < nch)
        def _():
            wait(1)
            process(1)

        return c

    lax.fori_loop(0, (nch + 1) // 2, pair, 0)
    pltpu.sync_copy(slab.at[pl.ds(sbase, RPT)], y.at[pl.ds(lo8, RPT)])


def _dis_body(c_ref, o_ref):
    o_ref[...] = lax.rsqrt(c_ref[...] + 1.0)


def _dis(cnt):
    o = pl.pallas_call(
        _dis_body,
        out_shape=jax.ShapeDtypeStruct((NPAD // D, D), jnp.float32),
    )(cnt.reshape(NPAD // D, D))
    return o.reshape(NPAD, 1)


def _mm_body(x_ref, w_ref, b_ref, pre_ref, post_ref, o_ref):
    xb = x_ref[...] * pre_ref[...] + b_ref[...]
    acc = jnp.dot(xb, w_ref[...], preferred_element_type=jnp.float32)
    o_ref[...] = acc * post_ref[...]


def _mm(xp, W, b2, pre, post):
    return pl.pallas_call(
        _mm_body,
        grid=(NPAD // BN,),
        in_specs=[
            pl.BlockSpec((BN, D), lambda i: (i, 0)),
            pl.BlockSpec((D, D), lambda i: (0, 0)),
            pl.BlockSpec((1, D), lambda i: (0, 0)),
            pl.BlockSpec((BN, 1), lambda i: (i, 0)),
            pl.BlockSpec((BN, 1), lambda i: (i, 0)),
        ],
        out_specs=pl.BlockSpec((BN, D), lambda i: (i, 0)),
        out_shape=jax.ShapeDtypeStruct((NPAD, D), jnp.float32),
    )(xp, W, b2, pre, post)


def _fin_body(y_ref, s_ref, b_ref, o_ref):
    o_ref[...] = y_ref[...] * s_ref[...] + b_ref[...]


def _fin(y, dis2, b2):
    return pl.pallas_call(
        _fin_body,
        grid=(NPAD // BN,),
        in_specs=[
            pl.BlockSpec((BN, D), lambda i: (i, 0)),
            pl.BlockSpec((BN, 1), lambda i: (i, 0)),
            pl.BlockSpec((1, D), lambda i: (0, 0)),
        ],
        out_specs=pl.BlockSpec((BN, D), lambda i: (i, 0)),
        out_shape=jax.ShapeDtypeStruct((NPAD, D), jnp.float32),
    )(y, dis2, b2)


def kernel(x, edge_index, W0, b0, W1, b1):
    xp = jnp.concatenate([x, jnp.zeros((NPAD - N, D), x.dtype)], axis=0)
    elist, meta, cnt = _build_lists(edge_index[0], edge_index[1])
    dis2 = _dis(cnt)
    ones = jnp.ones((NPAD, 1), jnp.float32)
    zb = jnp.zeros((1, D), jnp.float32)
    h1 = _mm(xp, W0, zb, ones, dis2)
    y1 = _aggregate(h1, elist, meta)
    h2 = _mm(y1, W1, b0.reshape(1, D), dis2, dis2)
    y2 = _aggregate(h2, elist, meta)
    out = _fin(y2, dis2, b1.reshape(1, D))
    return out[:N]


# speedup vs baseline: 14.6654x; 1.0470x over previous
import functools

import jax
import jax.numpy as jnp
from jax import lax
from jax.experimental import pallas as pl
from jax.experimental.pallas import tpu as pltpu
from jax.experimental.pallas import tpu_sc as plsc

N = 10000
E = 320000
D = 128
L = 16
NC, NS = 2, 16
NW = NC * NS
RPT = 320
NPAD = RPT * NW
DUMP = RPT
SMASK = (1 << 14) - 1
K = 128
CE = 4000
SCAP = 2048
SSZ = 8192
SMSK = SSZ - 1
EPAD = E + 4 * SCAP
SLABR = RPT + 8
CSL = RPT + L
BN = 1024

_MESH = plsc.VectorSubcoreMesh(
    core_axis_name="c", subcore_axis_name="s", num_cores=NC, num_subcores=NS)


@functools.partial(
    pl.kernel,
    out_type=(
        jax.ShapeDtypeStruct((NW * EPAD,), jnp.int32),
        jax.ShapeDtypeStruct((NW * L,), jnp.int32),
        jax.ShapeDtypeStruct((NPAD,), jnp.float32),
    ),
    mesh=_MESH,
    compiler_params=pltpu.CompilerParams(needs_layout_passes=False),
    scratch_types=(
        pltpu.VMEM((CE,), jnp.int32), pltpu.VMEM((CE,), jnp.int32),
        pltpu.VMEM((CE,), jnp.int32), pltpu.VMEM((CE,), jnp.int32),
        pltpu.VMEM((SSZ,), jnp.int32),
        pltpu.VMEM((RPT + L,), jnp.float32),
        pltpu.VMEM_SHARED((NS * CSL,), jnp.float32),
        pltpu.VMEM((K,), jnp.int32),
        pltpu.VMEM((K,), jnp.float32),
        pltpu.VMEM((L,), jnp.int32),
        pltpu.SemaphoreType.DMA, pltpu.SemaphoreType.DMA,
    ),
)
def _build_lists(esrc, edst, elist, meta, cnt, sb0, sb1, db0, db1, sstage,
                 zbuf, cshared, cdlrow, ones, mbuf, e0, e1):
    wid = lax.axis_index("s") * NC + lax.axis_index("c")
    lo = wid * RPT
    lbase = wid * EPAD
    cbase = pl.multiple_of(lax.axis_index("s") * CSL, 8)
    lanes = lax.iota(jnp.int32, L)

    for r in range(RPT // L + 1):
        zbuf[pl.ds(r * L, L)] = jnp.zeros((L,), jnp.float32)
    for r in range(K // L):
        ones[pl.ds(r * L, L)] = jnp.ones((L,), jnp.float32)
    pltpu.sync_copy(zbuf, cshared.at[pl.ds(cbase, RPT + L)])

    sb = (sb0, sb1)
    db = (db0, db1)
    sem = (e0, e1)

    def efire(c, b):
        eb = pl.multiple_of(c * CE, 8)
        pltpu.async_copy(esrc.at[pl.ds(eb, CE)], sb[b], sem[b])
        pltpu.async_copy(edst.at[pl.ds(eb, CE)], db[b], sem[b])

    def ewait(c, b):
        eb = pl.multiple_of(c * CE, 8)
        pltpu.make_async_copy(esrc.at[pl.ds(eb, CE)], sb[b], sem[b]).wait()
        pltpu.make_async_copy(edst.at[pl.ds(eb, CE)], db[b], sem[b]).wait()

    def flush_window(hbm_off, nsub):
        wb = pl.multiple_of(hbm_off & SMSK, 8)
        ho = pl.multiple_of(lbase + hbm_off, 8)
        pltpu.sync_copy(sstage.at[pl.ds(wb, SCAP)], elist.at[pl.ds(ho, SCAP)])

        def csub(k2, cc):
            for g in range(K // L):
                cdlrow[pl.ds(g * L, L)] = (
                    sstage[pl.ds(wb + k2 * K + g * L, L)] >> 14) + cbase
            pltpu.sync_copy(ones, cshared.at[cdlrow], add=True)
            return cc
        lax.fori_loop(0, nsub, csub, 0)

    def group(i, off, b):
        s16 = sb[b][pl.ds(i * L, L)]
        d16 = db[b][pl.ds(i * L, L)]
        m = (d16 >= lo) & (d16 < lo + RPT)
        packed = s16 | ((d16 - lo) << 14)
        cum = plsc.cumsum(m.astype(jnp.int32))
        plsc.store_scatter(sstage, [(off + cum - 1) & SMSK], packed, mask=m)
        return off + cum[L - 1]

    def scan_chunk(c, carry, b):
        off, hbm_off = carry
        ewait(c, b)
        off = lax.fori_loop(0, CE // L, lambda i, o: group(i, o, b), off,
                            unroll=4)

        @pl.when(c + 2 < E // CE)
        def _():
            efire(c + 2, b)

        for _ in range(2):
            do_f = off - hbm_off >= SCAP

            @pl.when(do_f)
            def _(ho=hbm_off):
                flush_window(ho, 16)
            hbm_off = jnp.where(do_f, hbm_off + SCAP, hbm_off)
        return off, hbm_off

    def pair(p, carry):
        carry = scan_chunk(p * 2, carry, 0)
        carry = scan_chunk(p * 2 + 1, carry, 1)
        return carry

    efire(0, 0)
    efire(1, 1)
    off, hbm_off = lax.fori_loop(0, E // CE // 2, pair,
                                 (jnp.int32(0), jnp.int32(0)))

    total = off
    padv = jnp.full((L,), DUMP << 14, jnp.int32)
    for j in range(K // L):
        plsc.store_scatter(sstage, [(off + j * L + lanes) & SMSK], padv)
    nch = (total + (K - 1)) // K
    ptotal = nch * K
    for _ in range(4):
        do_f = hbm_off < ptotal

        @pl.when(do_f)
        def _(ho=hbm_off):
            flush_window(ho, jnp.minimum((ptotal - ho) // K, 16))
        hbm_off = jnp.where(do_f, hbm_off + SCAP, hbm_off)

    mv = jnp.where(lanes == 1, nch, total)
    mbuf[...] = mv
    pltpu.sync_copy(mbuf, meta.at[pl.ds(pl.multiple_of(wid * L, 8), L)])
    pltpu.sync_copy(cshared.at[pl.ds(cbase, RPT)], zbuf.at[pl.ds(0, RPT)])
    pltpu.sync_copy(zbuf.at[pl.ds(0, RPT)],
                    cnt.at[pl.ds(pl.multiple_of(lo, 8), RPT)])


@functools.partial(
    pl.kernel,
    out_type=jax.ShapeDtypeStruct((NPAD, D), jnp.float32),
    mesh=_MESH,
    compiler_params=pltpu.CompilerParams(needs_layout_passes=False),
    scratch_types=(
        pltpu.VMEM_SHARED((NS * SLABR, D), jnp.float32),
        pltpu.VMEM((K,), jnp.int32), pltpu.VMEM((K,), jnp.int32),
        pltpu.VMEM((K,), jnp.int32),
        pltpu.VMEM((K,), jnp.int32), pltpu.VMEM((K,), jnp.int32),
        pltpu.VMEM((K,), jnp.int32),
        pltpu.VMEM((K,), jnp.int32), pltpu.VMEM((K,), jnp.int32),
        pltpu.VMEM((K,), jnp.int32),
        pltpu.VMEM((K, D), jnp.float32), pltpu.VMEM((K, D), jnp.float32),
        pltpu.VMEM((K, D), jnp.float32),
        pltpu.VMEM((L,), jnp.int32),
        pltpu.SemaphoreType.DMA, pltpu.SemaphoreType.DMA,
        pltpu.SemaphoreType.DMA,
        pltpu.SemaphoreType.DMA, pltpu.SemaphoreType.DMA,
        pltpu.SemaphoreType.DMA,
        pltpu.SemaphoreType.DMA, pltpu.SemaphoreType.DMA,
        pltpu.SemaphoreType.DMA,
    ),
)
def _aggregate(hp, elist, meta, y, slab, p0, p1, p2, i0, i1, i2,
               d0, d1, d2, r0, r1, r2, mbuf,
               l0, l1, l2, g0, g1, g2, sc0, sc1, sc2):
    wid = lax.axis_index("s") * NC + lax.axis_index("c")
    lo = wid * RPT
    lbase = wid * EPAD
    lo8 = pl.multiple_of(lo, 8)
    sbase = pl.multiple_of(lax.axis_index("s") * SLABR, 8)
    pltpu.sync_copy(meta.at[pl.ds(pl.multiple_of(wid * L, 8), L)], mbuf)
    nch = mbuf[...][1]
    pltpu.sync_copy(hp.at[pl.ds(lo8, RPT)], slab.at[pl.ds(sbase, RPT)])

    pb = (p0, p1, p2)
    ib = (i0, i1, i2)
    dbf = (d0, d1, d2)
    rb = (r0, r1, r2)
    lsem = (l0, l1, l2)
    gsem = (g0, g1, g2)
    ssem = (sc0, sc1, sc2)

    def lfire(c, b):
        co = pl.multiple_of(lbase + c * K, 8)
        pltpu.async_copy(elist.at[pl.ds(co, K)], pb[b], lsem[b])

    def lwait(c, b):
        co = pl.multiple_of(lbase + c * K, 8)
        pltpu.make_async_copy(elist.at[pl.ds(co, K)], pb[b], lsem[b]).wait()

    def swait(b):
        pltpu.make_async_copy(rb[b], slab.at[dbf[b]], ssem[b]).wait()

    def prep(c, b):
        lwait(c, b)
        for g in range(K // L):
            w = pb[b][pl.ds(g * L, L)]
            ib[b][pl.ds(g * L, L)] = w & SMASK
            dbf[b][pl.ds(g * L, L)] = (w >> 14) + sbase
        pltpu.async_copy(hp.at[ib[b]], rb[b], gsem[b])

        @pl.when(c + 3 < nch)
        def _():
            lfire(c + 3, b)

    def consume(c, b):
        pltpu.make_async_copy(hp.at[ib[b]], rb[b], gsem[b]).wait()
        pltpu.sync_copy(rb[b], slab.at[dbf[b]], add=True)

    for c in range(3):
        @pl.when(c < nch)
        def _(c=c):
            lfire(c, c)

    @pl.when(0 < nch)
    def _():
        prep(0, 0)

    def triple(t, carry):
        c0 = t * 3
        for (po, co_) in ((1, 0), (2, 1), (3, 2)):
            pc = c0 + po
            cc = c0 + co_

            @pl.when(pc < nch)
            def _(pc=pc, b=po % 3):
                prep(pc, b)

            @pl.when(cc < nch)
            def _(cc=cc, b=co_ % 3):
                consume(cc, b)
        return carry

    lax.fori_loop(0, (nch + 2) // 3, triple, 0)
    pltpu.sync_copy(slab.at[pl.ds(sbase, RPT)], y.at[pl.ds(lo8, RPT)])


def _dis_body(c_ref, o_ref):
    o_ref[...] = lax.rsqrt(c_ref[...] + 1.0)


def _dis(cnt):
    o = pl.pallas_call(
        _dis_body,
        out_shape=jax.ShapeDtypeStruct((NPAD // D, D), jnp.float32),
    )(cnt.reshape(NPAD // D, D))
    return o.reshape(NPAD, 1)


def _mm_body(x_ref, w_ref, b_ref, pre_ref, post_ref, o_ref):
    xb = x_ref[...] * pre_ref[...] + b_ref[...]
    acc = jnp.dot(xb, w_ref[...], preferred_element_type=jnp.float32)
    o_ref[...] = acc * post_ref[...]


def _mm(xp, W, b2, pre, post):
    return pl.pallas_call(
        _mm_body,
        grid=(NPAD // BN,),
        in_specs=[
            pl.BlockSpec((BN, D), lambda i: (i, 0)),
            pl.BlockSpec((D, D), lambda i: (0, 0)),
            pl.BlockSpec((1, D), lambda i: (0, 0)),
            pl.BlockSpec((BN, 1), lambda i: (i, 0)),
            pl.BlockSpec((BN, 1), lambda i: (i, 0)),
        ],
        out_specs=pl.BlockSpec((BN, D), lambda i: (i, 0)),
        out_shape=jax.ShapeDtypeStruct((NPAD, D), jnp.float32),
    )(xp, W, b2, pre, post)


def _fin_body(y_ref, s_ref, b_ref, o_ref):
    o_ref[...] = y_ref[...] * s_ref[...] + b_ref[...]


def _fin(y, dis2, b2):
    return pl.pallas_call(
        _fin_body,
        grid=(NPAD // BN,),
        in_specs=[
            pl.BlockSpec((BN, D), lambda i: (i, 0)),
            pl.BlockSpec((BN, 1), lambda i: (i, 0)),
            pl.BlockSpec((1, D), lambda i: (0, 0)),
        ],
        out_specs=pl.BlockSpec((BN, D), lambda i: (i, 0)),
        out_shape=jax.ShapeDtypeStruct((NPAD, D), jnp.float32),
    )(y, dis2, b2)


def kernel(x, edge_index, W0, b0, W1, b1):
    xp = jnp.concatenate([x, jnp.zeros((NPAD - N, D), x.dtype)], axis=0)
    elist, meta, cnt = _build_lists(edge_index[0], edge_index[1])
    dis2 = _dis(cnt)
    ones = jnp.ones((NPAD, 1), jnp.float32)
    zb = jnp.zeros((1, D), jnp.float32)
    h1 = _mm(xp, W0, zb, ones, dis2)
    y1 = _aggregate(h1, elist, meta)
    h2 = _mm(y1, W1, b0.reshape(1, D), dis2, dis2)
    y2 = _aggregate(h2, elist, meta)
    out = _fin(y2, dis2, b1.reshape(1, D))
    return out[:N]
